# baseline (speedup 1.0000x reference)
import jax
import jax.numpy as jnp
from jax import lax
from jax.experimental import pallas as pl
from jax.experimental.pallas import tpu as pltpu
from jax.experimental.pallas import tpu_sc as plsc

NU = 25000
NV = 25000
DIN = 128
H0 = 128
HALF = 64
FH = 64
EMB = 64
NSUP = 5
EPC = 80000
NPAIR = 100000
RB = 1000

ECH = 128
NCHUNK = NSUP * EPC // ECH
CPC = EPC // ECH
GSZ = 5
NGRP = 39
NLEFT = NCHUNK - 16 * NGRP * GSZ
CLR = 1000


def _proj_body(u_ref, v_ref, wg_ref, tu_ref, tv_ref):
    wacc = jnp.zeros((DIN, H0), jnp.float32)
    for r in range(NSUP):
        wacc = wacc + wg_ref[r]
        tu_ref[r] = jnp.dot(u_ref[...], wacc, preferred_element_type=jnp.float32)
        tv_ref[r] = jnp.dot(v_ref[...], wacc, preferred_element_type=jnp.float32)


def _project(u_features, v_features, W_gcn):
    nblk = NU // RB
    return pl.pallas_call(
        _proj_body,
        grid=(nblk,),
        in_specs=[
            pl.BlockSpec((RB, DIN), lambda j: (j, 0)),
            pl.BlockSpec((RB, DIN), lambda j: (j, 0)),
            pl.BlockSpec((NSUP, DIN, H0), lambda j: (0, 0, 0)),
        ],
        out_specs=[
            pl.BlockSpec((NSUP, RB, H0), lambda j: (0, j, 0)),
            pl.BlockSpec((NSUP, RB, H0), lambda j: (0, j, 0)),
        ],
        out_shape=[
            jax.ShapeDtypeStruct((NSUP, NU, H0), jnp.float32),
            jax.ShapeDtypeStruct((NSUP, NV, H0), jnp.float32),
        ],
    )(u_features, v_features, W_gcn)


def _mp_body(eu2_hbm, ev2_hbm, evals2_hbm, tmpu_hbm, tmpv_hbm, zeros_hbm,
             zu_out, zv_out,
             acc, rows, msg, idx_g2, idx_d2, evals2,
             semL, semG):
    core = lax.axis_index("c")
    tid = lax.axis_index("s")
    col0 = core * HALF

    def clear_acc():
        for k in range(2):
            c = tid + 16 * k

            @pl.when(c < NU // CLR)
            def _():
                pltpu.sync_copy(zeros_hbm.at[pl.ds(c * CLR, CLR)],
                                acc.at[pl.ds(c * CLR, CLR)])

    def dump(out_z):
        for k in range(2):
            c = tid + 16 * k

            @pl.when(c < NU // CLR)
            def _():
                @pl.when(core == 0)
                def _():
                    pltpu.sync_copy(acc.at[pl.ds(c * CLR, CLR)],
                                    out_z.at[pl.ds(c * CLR, CLR),
                                             pl.ds(0, HALF)])

                @pl.when(core == 1)
                def _():
                    pltpu.sync_copy(acc.at[pl.ds(c * CLR, CLR)],
                                    out_z.at[pl.ds(c * CLR, CLR),
                                             pl.ds(HALF, HALF)])

    def scale(cur, mg, j):
        @plsc.parallel_loop(0, ECH // 16, unroll=2)
        def _(g):
            e16 = evals2[j, pl.ds(g * 16, 16)]
            for jj in range(16):
                s = e16.at[jnp.full((16,), jj, jnp.int32)].get(
                    mode="promise_in_bounds")
                e = g * 16 + jj
                for q in range(HALF // 16):
                    mg[e, pl.ds(q * 16, 16)] = (
                        cur[e, pl.ds(col0 + q * 16, 16)] * s)

    def off_add(j, cglob):
        off = (cglob // CPC) * NU
        for g in range(ECH // 16):
            idx_g2[j, pl.ds(g * 16, 16)] = idx_g2[j, pl.ds(g * 16, 16)] + off

    def do_phase(tbl, gidx2, sidx2):
        def group(gq, cg):
            c0 = (tid * NGRP + gq) * GSZ
            h1 = pltpu.async_copy(gidx2.at[pl.ds(c0, GSZ)], idx_g2, semL)
            h2 = pltpu.async_copy(sidx2.at[pl.ds(c0, GSZ)], idx_d2, semL)
            h3 = pltpu.async_copy(evals2_hbm.at[pl.ds(c0, GSZ)], evals2, semL)
            h1.wait()
            h2.wait()
            h3.wait()

            def oadd(j, c):
                off_add(j, c0 + j)
                return c
            lax.fori_loop(0, GSZ, oadd, 0)

            pltpu.async_copy(tbl.at[idx_g2.at[0]], rows, semG)

            def chunk_j(j, c):
                pltpu.make_async_copy(tbl.at[idx_g2.at[j]], rows, semG).wait()
                scale(rows, msg, j)

                @pl.when(j < GSZ - 1)
                def _():
                    pltpu.async_copy(tbl.at[idx_g2.at[j + 1]], rows, semG)

                pltpu.sync_copy(msg, acc.at[idx_d2.at[j]], add=True)
                return c
            lax.fori_loop(0, GSZ, chunk_j, 0)
            return cg
        lax.fori_loop(0, NGRP, group, 0)

        @pl.when(tid < NLEFT)
        def _():
            row = 16 * NGRP * GSZ + tid
            pltpu.sync_copy(gidx2.at[pl.ds(row, 1)], idx_g2.at[pl.ds(0, 1)])
            pltpu.sync_copy(sidx2.at[pl.ds(row, 1)], idx_d2.at[pl.ds(0, 1)])
            pltpu.sync_copy(evals2_hbm.at[pl.ds(row, 1)],
                            evals2.at[pl.ds(0, 1)])
            off_add(0, row)
            pltpu.async_copy(tbl.at[idx_g2.at[0]], rows, semG).wait()
            scale(rows, msg, 0)
            pltpu.sync_copy(msg, acc.at[idx_d2.at[0]], add=True)

    clear_acc()
    plsc.subcore_barrier()
    do_phase(tmpv_hbm, ev2_hbm, eu2_hbm)
    plsc.subcore_barrier()
    dump(zu_out)
    plsc.subcore_barrier()
    clear_acc()
    plsc.subcore_barrier()
    do_phase(tmpu_hbm, eu2_hbm, ev2_hbm)
    plsc.subcore_barrier()
    dump(zv_out)


def _message_passing(eu, ev, evals, tmp_u_flat, tmp_v_flat):
    mesh = plsc.VectorSubcoreMesh(core_axis_name="c", subcore_axis_name="s")
    zshape = jax.ShapeDtypeStruct((NU, H0), jnp.float32)
    eu2 = eu.reshape(NCHUNK, ECH)
    ev2 = ev.reshape(NCHUNK, ECH)
    evals2d = evals.reshape(NCHUNK, ECH)
    zeros = jnp.zeros((NU, HALF), jnp.float32)
    return pl.kernel(
        _mp_body,
        out_type=[zshape, zshape],
        mesh=mesh,
        scratch_types=[
            pltpu.VMEM_SHARED((NU, HALF), jnp.float32),
            pltpu.VMEM((ECH, H0), jnp.float32),
            pltpu.VMEM((ECH, HALF), jnp.float32),
            pltpu.VMEM((GSZ, ECH), jnp.int32),
            pltpu.VMEM((GSZ, ECH), jnp.int32),
            pltpu.VMEM((GSZ, ECH), jnp.float32),
            pltpu.SemaphoreType.DMA,
            pltpu.SemaphoreType.DMA,
        ],
        compiler_params=pltpu.CompilerParams(use_tc_tiling_on_sc=False),
    )(eu2, ev2, evals2d, tmp_u_flat, tmp_v_flat, zeros)


def _comb_body(zu, zv, su, sv, wfu, bfu, wfv, bfv, wdu, wdv, out_ref):
    def side(s_ref, wf, bf):
        t = jnp.dot(s_ref[...], wf[...], preferred_element_type=jnp.float32)
        return jnp.maximum(t + bf[...], 0.0)

    def emb(z, f, wd):
        w = wd[...]
        return (jnp.dot(jnp.maximum(z[...], 0.0), w[:H0],
                        preferred_element_type=jnp.float32)
                + jnp.dot(f, w[H0:], preferred_element_type=jnp.float32))

    out_ref[:, :EMB] = emb(zu, side(su, wfu, bfu), wdu)
    out_ref[:, EMB:] = emb(zv, side(sv, wfv, bfv), wdv)


def _combine(zu, zv, su, sv, wfu, bfu, wfv, bfv, wdu, wdv):
    nblk = NU // RB
    zspec = pl.BlockSpec((RB, H0), lambda j: (j, 0))
    sspec = pl.BlockSpec((RB, 32), lambda j: (j, 0))
    wfspec = pl.BlockSpec((32, FH), lambda j: (0, 0))
    bspec = pl.BlockSpec((1, FH), lambda j: (0, 0))
    wdspec = pl.BlockSpec((H0 + FH, EMB), lambda j: (0, 0))
    return pl.pallas_call(
        _comb_body,
        grid=(nblk,),
        in_specs=[zspec, zspec, sspec, sspec,
                  wfspec, bspec, wfspec, bspec, wdspec, wdspec],
        out_specs=pl.BlockSpec((RB, 2 * EMB), lambda j: (j, 0)),
        out_shape=jax.ShapeDtypeStruct((NU, 2 * EMB), jnp.float32),
    )(zu, zv, su, sv, wfu, bfu.reshape(1, FH), wfv, bfv.reshape(1, FH),
      wdu, wdv)


PCH = 80
NPCH = NPAIR // PCH


def _pairs_body(uidx_hbm, vidx_hbm, emb_hbm, gu_hbm, gv_hbm,
                idxb, rowsb, sem):
    core = lax.axis_index("c")
    tid = lax.axis_index("s")
    wid = tid * 2 + core
    nk = 39 + jnp.where(wid < NPCH - 39 * 32, 1, 0)

    def chunk(kk, c):
        cidx = wid + 32 * kk
        base = cidx * PCH
        pltpu.sync_copy(uidx_hbm.at[pl.ds(base, PCH)], idxb)
        pltpu.async_copy(emb_hbm.at[idxb], rowsb, sem).wait()
        pltpu.sync_copy(rowsb, gu_hbm.at[pl.ds(base, PCH)])
        pltpu.sync_copy(vidx_hbm.at[pl.ds(base, PCH)], idxb)
        pltpu.async_copy(emb_hbm.at[idxb], rowsb, sem).wait()
        pltpu.sync_copy(rowsb, gv_hbm.at[pl.ds(base, PCH)])
        return c
    lax.fori_loop(0, nk, chunk, 0)


def _gather_pairs(ui, vi, emb_cat):
    mesh = plsc.VectorSubcoreMesh(core_axis_name="c", subcore_axis_name="s")
    gshape = jax.ShapeDtypeStruct((NPAIR, 2 * EMB), jnp.float32)
    return pl.kernel(
        _pairs_body,
        out_type=[gshape, gshape],
        mesh=mesh,
        scratch_types=[
            pltpu.VMEM((PCH,), jnp.int32),
            pltpu.VMEM((PCH, 2 * EMB), jnp.float32),
            pltpu.SemaphoreType.DMA,
        ],
        compiler_params=pltpu.CompilerParams(use_tc_tiling_on_sc=False),
    )(ui, vi, emb_cat)


PB = 2000


def _dec_body(gu_ref, gv_ref, p_ref, wc_ref, out_ref):
    gu = gu_ref[:, :EMB]
    gv = gv_ref[:, EMB:]
    b0 = jnp.sum(jnp.dot(gu, p_ref[0], preferred_element_type=jnp.float32)
                 * gv, axis=1)
    b1 = jnp.sum(jnp.dot(gu, p_ref[1], preferred_element_type=jnp.float32)
                 * gv, axis=1)
    out_ref[...] = (b0[:, None] * wc_ref[0][None, :]
                    + b1[:, None] * wc_ref[1][None, :])


def _decode(gu, gv, P_basis, W_comb):
    nblk = NPAIR // PB
    ncls = W_comb.shape[1]
    return pl.pallas_call(
        _dec_body,
        grid=(nblk,),
        in_specs=[
            pl.BlockSpec((PB, 2 * EMB), lambda j: (j, 0)),
            pl.BlockSpec((PB, 2 * EMB), lambda j: (j, 0)),
            pl.BlockSpec((2, EMB, EMB), lambda j: (0, 0, 0)),
            pl.BlockSpec((2, ncls), lambda j: (0, 0)),
        ],
        out_specs=pl.BlockSpec((PB, ncls), lambda j: (j, 0)),
        out_shape=jax.ShapeDtypeStruct((NPAIR, ncls), jnp.float32),
    )(gu, gv, P_basis, W_comb)


def kernel(u_features, v_features, u_features_side, v_features_side,
           edge_u, edge_v, edge_vals, u_indices, v_indices,
           W_gcn, W_feat_u, b_feat_u, W_feat_v, b_feat_v,
           W_dense_u, W_dense_v, P_basis, W_comb):
    eu = edge_u.astype(jnp.int32)
    ev = edge_v.astype(jnp.int32)
    ui = u_indices.astype(jnp.int32)
    vi = v_indices.astype(jnp.int32)

    tmp_u, tmp_v = _project(u_features, v_features, W_gcn)
    tmp_u_flat = tmp_u.reshape(NSUP * NU, H0)
    tmp_v_flat = tmp_v.reshape(NSUP * NV, H0)

    zu, zv = _message_passing(eu, ev, edge_vals, tmp_u_flat, tmp_v_flat)

    emb_cat = _combine(zu, zv, u_features_side, v_features_side,
                       W_feat_u, b_feat_u, W_feat_v, b_feat_v,
                       W_dense_u, W_dense_v)

    gu, gv = _gather_pairs(ui, vi, emb_cat)
    return _decode(gu, gv, P_basis, W_comb)


# speedup vs baseline: 1.5735x; 1.5735x over previous
import jax
import jax.numpy as jnp
from jax import lax
from jax.experimental import pallas as pl
from jax.experimental.pallas import tpu as pltpu
from jax.experimental.pallas import tpu_sc as plsc

NU = 25000
NV = 25000
DIN = 128
H0 = 128
HALF = 64
FH = 64
EMB = 64
NSUP = 5
EPC = 80000
NPAIR = 100000
RB = 1000

ECH = 128
NCHUNK = NSUP * EPC // ECH
CPC = EPC // ECH
GSZ = 13
NGRP = 15
NLEFT = NCHUNK - 16 * NGRP * GSZ
CLR = 1000


def _proj_body(u_ref, v_ref, wg_ref, tu_ref, tv_ref):
    wacc = jnp.zeros((DIN, H0), jnp.float32)
    for r in range(NSUP):
        wacc = wacc + wg_ref[r]
        tu_ref[r] = jnp.dot(u_ref[...], wacc, preferred_element_type=jnp.float32)
        tv_ref[r] = jnp.dot(v_ref[...], wacc, preferred_element_type=jnp.float32)


def _project(u_features, v_features, W_gcn):
    nblk = NU // RB
    return pl.pallas_call(
        _proj_body,
        grid=(nblk,),
        in_specs=[
            pl.BlockSpec((RB, DIN), lambda j: (j, 0)),
            pl.BlockSpec((RB, DIN), lambda j: (j, 0)),
            pl.BlockSpec((NSUP, DIN, H0), lambda j: (0, 0, 0)),
        ],
        out_specs=[
            pl.BlockSpec((NSUP, RB, H0), lambda j: (0, j, 0)),
            pl.BlockSpec((NSUP, RB, H0), lambda j: (0, j, 0)),
        ],
        out_shape=[
            jax.ShapeDtypeStruct((NSUP, NU, H0), jnp.float32),
            jax.ShapeDtypeStruct((NSUP, NV, H0), jnp.float32),
        ],
    )(u_features, v_features, W_gcn)


def _mp_body(eu2_hbm, ev2_hbm, evals2_hbm, tmpu_hbm, tmpv_hbm, zeros_hbm,
             zu_out, zv_out,
             acc, rows_a, rows_b, idx_g2, idx_d2, evals2,
             semL, semG, semS):
    core = lax.axis_index("c")
    tid = lax.axis_index("s")

    def clear_acc():
        for k in range(2):
            c = tid + 16 * k

            @pl.when(c < NU // CLR)
            def _():
                pltpu.sync_copy(zeros_hbm.at[pl.ds(c * CLR, CLR)],
                                acc.at[pl.ds(c * CLR, CLR)])

    def dump(out_z):
        for k in range(2):
            c = tid + 16 * k

            @pl.when(c < NU // CLR)
            def _():
                @pl.when(core == 0)
                def _():
                    pltpu.sync_copy(acc.at[pl.ds(c * CLR, CLR)],
                                    out_z.at[pl.ds(c * CLR, CLR),
                                             pl.ds(0, HALF)])

                @pl.when(core == 1)
                def _():
                    pltpu.sync_copy(acc.at[pl.ds(c * CLR, CLR)],
                                    out_z.at[pl.ds(c * CLR, CLR),
                                             pl.ds(HALF, HALF)])

    def scale(cur, j):
        @plsc.parallel_loop(0, ECH // 16, unroll=2)
        def _(g):
            e16 = evals2[j, pl.ds(g * 16, 16)]
            for jj in range(16):
                s = e16.at[jnp.full((16,), jj, jnp.int32)].get(
                    mode="promise_in_bounds")
                e = g * 16 + jj
                for q in range(HALF // 16):
                    cur[e, pl.ds(q * 16, 16)] = cur[e, pl.ds(q * 16, 16)] * s

    def off_add(j, cglob):
        off = 2 * (cglob // CPC) * NU + core
        for g in range(ECH // 16):
            idx_g2[j, pl.ds(g * 16, 16)] = (
                idx_g2[j, pl.ds(g * 16, 16)] * 2 + off)

    def do_phase(tbl, gidx2, sidx2):
        def group(gq, cg):
            c0 = (tid * NGRP + gq) * GSZ
            h1 = pltpu.async_copy(gidx2.at[pl.ds(c0, GSZ)], idx_g2, semL)
            h2 = pltpu.async_copy(sidx2.at[pl.ds(c0, GSZ)], idx_d2, semL)
            h3 = pltpu.async_copy(evals2_hbm.at[pl.ds(c0, GSZ)], evals2, semL)
            h1.wait()
            h2.wait()
            h3.wait()

            def oadd(j, c):
                off_add(j, c0 + j)
                return c
            lax.fori_loop(0, GSZ, oadd, 0)

            pltpu.async_copy(tbl.at[idx_g2.at[0]], rows_a, semG)

            def run(j, cur, oth):
                pltpu.make_async_copy(tbl.at[idx_g2.at[j]], cur, semG).wait()

                @pl.when(j > 0)
                def _():
                    pltpu.make_async_copy(
                        oth, acc.at[idx_d2.at[j - 1]], semS).wait()

                @pl.when(j < GSZ - 1)
                def _():
                    pltpu.async_copy(tbl.at[idx_g2.at[j + 1]], oth, semG)

                scale(cur, j)
                pltpu.async_copy(cur, acc.at[idx_d2.at[j]], semS, add=True)

            def chunk_j(j, c):
                @pl.when(j % 2 == 0)
                def _():
                    run(j, rows_a, rows_b)

                @pl.when(j % 2 == 1)
                def _():
                    run(j, rows_b, rows_a)
                return c
            lax.fori_loop(0, GSZ, chunk_j, 0)
            pltpu.make_async_copy(
                rows_a, acc.at[idx_d2.at[GSZ - 1]], semS).wait()
            return cg
        lax.fori_loop(0, NGRP, group, 0)

        @pl.when(tid < NLEFT)
        def _():
            row = 16 * NGRP * GSZ + tid
            pltpu.sync_copy(gidx2.at[pl.ds(row, 1)], idx_g2.at[pl.ds(0, 1)])
            pltpu.sync_copy(sidx2.at[pl.ds(row, 1)], idx_d2.at[pl.ds(0, 1)])
            pltpu.sync_copy(evals2_hbm.at[pl.ds(row, 1)],
                            evals2.at[pl.ds(0, 1)])
            off_add(0, row)
            pltpu.async_copy(tbl.at[idx_g2.at[0]], rows_a, semG).wait()
            scale(rows_a, 0)
            pltpu.sync_copy(rows_a, acc.at[idx_d2.at[0]], add=True)

    clear_acc()
    plsc.subcore_barrier()
    do_phase(tmpv_hbm, ev2_hbm, eu2_hbm)
    plsc.subcore_barrier()
    dump(zu_out)
    plsc.subcore_barrier()
    clear_acc()
    plsc.subcore_barrier()
    do_phase(tmpu_hbm, eu2_hbm, ev2_hbm)
    plsc.subcore_barrier()
    dump(zv_out)


def _message_passing(eu, ev, evals, tmp_u_flat, tmp_v_flat):
    mesh = plsc.VectorSubcoreMesh(core_axis_name="c", subcore_axis_name="s")
    zshape = jax.ShapeDtypeStruct((NU, H0), jnp.float32)
    eu2 = eu.reshape(NCHUNK, ECH)
    ev2 = ev.reshape(NCHUNK, ECH)
    evals2d = evals.reshape(NCHUNK, ECH)
    zeros = jnp.zeros((NU, HALF), jnp.float32)
    return pl.kernel(
        _mp_body,
        out_type=[zshape, zshape],
        mesh=mesh,
        scratch_types=[
            pltpu.VMEM_SHARED((NU, HALF), jnp.float32),
            pltpu.VMEM((ECH, HALF), jnp.float32),
            pltpu.VMEM((ECH, HALF), jnp.float32),
            pltpu.VMEM((GSZ, ECH), jnp.int32),
            pltpu.VMEM((GSZ, ECH), jnp.int32),
            pltpu.VMEM((GSZ, ECH), jnp.float32),
            pltpu.SemaphoreType.DMA,
            pltpu.SemaphoreType.DMA,
            pltpu.SemaphoreType.DMA,
        ],
        compiler_params=pltpu.CompilerParams(use_tc_tiling_on_sc=False),
    )(eu2, ev2, evals2d, tmp_u_flat, tmp_v_flat, zeros)


def _comb_body(zu, zv, su, sv, wfu, bfu, wfv, bfv, wdu, wdv, out_ref):
    def side(s_ref, wf, bf):
        t = jnp.dot(s_ref[...], wf[...], preferred_element_type=jnp.float32)
        return jnp.maximum(t + bf[...], 0.0)

    def emb(z, f, wd):
        w = wd[...]
        return (jnp.dot(jnp.maximum(z[...], 0.0), w[:H0],
                        preferred_element_type=jnp.float32)
                + jnp.dot(f, w[H0:], preferred_element_type=jnp.float32))

    out_ref[:, :EMB] = emb(zu, side(su, wfu, bfu), wdu)
    out_ref[:, EMB:] = emb(zv, side(sv, wfv, bfv), wdv)


def _combine(zu, zv, su, sv, wfu, bfu, wfv, bfv, wdu, wdv):
    nblk = NU // RB
    zspec = pl.BlockSpec((RB, H0), lambda j: (j, 0))
    sspec = pl.BlockSpec((RB, 32), lambda j: (j, 0))
    wfspec = pl.BlockSpec((32, FH), lambda j: (0, 0))
    bspec = pl.BlockSpec((1, FH), lambda j: (0, 0))
    wdspec = pl.BlockSpec((H0 + FH, EMB), lambda j: (0, 0))
    return pl.pallas_call(
        _comb_body,
        grid=(nblk,),
        in_specs=[zspec, zspec, sspec, sspec,
                  wfspec, bspec, wfspec, bspec, wdspec, wdspec],
        out_specs=pl.BlockSpec((RB, 2 * EMB), lambda j: (j, 0)),
        out_shape=jax.ShapeDtypeStruct((NU, 2 * EMB), jnp.float32),
    )(zu, zv, su, sv, wfu, bfu.reshape(1, FH), wfv, bfv.reshape(1, FH),
      wdu, wdv)


PCH = 80
NPCH = NPAIR // PCH


def _pairs_body(uidx_hbm, vidx_hbm, emb_hbm, gu_hbm, gv_hbm,
                idxb, rowsb, sem):
    core = lax.axis_index("c")
    tid = lax.axis_index("s")
    wid = tid * 2 + core
    nk = 39 + jnp.where(wid < NPCH - 39 * 32, 1, 0)

    def chunk(kk, c):
        cidx = wid + 32 * kk
        base = cidx * PCH
        pltpu.sync_copy(uidx_hbm.at[pl.ds(base, PCH)], idxb)
        pltpu.async_copy(emb_hbm.at[idxb], rowsb, sem).wait()
        pltpu.sync_copy(rowsb, gu_hbm.at[pl.ds(base, PCH)])
        pltpu.sync_copy(vidx_hbm.at[pl.ds(base, PCH)], idxb)
        pltpu.async_copy(emb_hbm.at[idxb], rowsb, sem).wait()
        pltpu.sync_copy(rowsb, gv_hbm.at[pl.ds(base, PCH)])
        return c
    lax.fori_loop(0, nk, chunk, 0)


def _gather_pairs(ui, vi, emb_cat):
    mesh = plsc.VectorSubcoreMesh(core_axis_name="c", subcore_axis_name="s")
    gshape = jax.ShapeDtypeStruct((NPAIR, 2 * EMB), jnp.float32)
    return pl.kernel(
        _pairs_body,
        out_type=[gshape, gshape],
        mesh=mesh,
        scratch_types=[
            pltpu.VMEM((PCH,), jnp.int32),
            pltpu.VMEM((PCH, 2 * EMB), jnp.float32),
            pltpu.SemaphoreType.DMA,
        ],
        compiler_params=pltpu.CompilerParams(use_tc_tiling_on_sc=False),
    )(ui, vi, emb_cat)


PB = 2000


def _dec_body(gu_ref, gv_ref, p_ref, wc_ref, out_ref):
    gu = gu_ref[:, :EMB]
    gv = gv_ref[:, EMB:]
    b0 = jnp.sum(jnp.dot(gu, p_ref[0], preferred_element_type=jnp.float32)
                 * gv, axis=1)
    b1 = jnp.sum(jnp.dot(gu, p_ref[1], preferred_element_type=jnp.float32)
                 * gv, axis=1)
    out_ref[...] = (b0[:, None] * wc_ref[0][None, :]
                    + b1[:, None] * wc_ref[1][None, :])


def _decode(gu, gv, P_basis, W_comb):
    nblk = NPAIR // PB
    ncls = W_comb.shape[1]
    return pl.pallas_call(
        _dec_body,
        grid=(nblk,),
        in_specs=[
            pl.BlockSpec((PB, 2 * EMB), lambda j: (j, 0)),
            pl.BlockSpec((PB, 2 * EMB), lambda j: (j, 0)),
            pl.BlockSpec((2, EMB, EMB), lambda j: (0, 0, 0)),
            pl.BlockSpec((2, ncls), lambda j: (0, 0)),
        ],
        out_specs=pl.BlockSpec((PB, ncls), lambda j: (j, 0)),
        out_shape=jax.ShapeDtypeStruct((NPAIR, ncls), jnp.float32),
    )(gu, gv, P_basis, W_comb)


def kernel(u_features, v_features, u_features_side, v_features_side,
           edge_u, edge_v, edge_vals, u_indices, v_indices,
           W_gcn, W_feat_u, b_feat_u, W_feat_v, b_feat_v,
           W_dense_u, W_dense_v, P_basis, W_comb):
    eu = edge_u.astype(jnp.int32)
    ev = edge_v.astype(jnp.int32)
    ui = u_indices.astype(jnp.int32)
    vi = v_indices.astype(jnp.int32)

    tmp_u, tmp_v = _project(u_features, v_features, W_gcn)
    tmp_u_flat = tmp_u.reshape(2 * NSUP * NU, HALF)
    tmp_v_flat = tmp_v.reshape(2 * NSUP * NV, HALF)

    zu, zv = _message_passing(eu, ev, edge_vals, tmp_u_flat, tmp_v_flat)

    emb_cat = _combine(zu, zv, u_features_side, v_features_side,
                       W_feat_u, b_feat_u, W_feat_v, b_feat_v,
                       W_dense_u, W_dense_v)

    gu, gv = _gather_pairs(ui, vi, emb_cat)
    return _decode(gu, gv, P_basis, W_comb)


# speedup vs baseline: 1.7775x; 1.1296x over previous
import jax
import jax.numpy as jnp
from jax import lax
from jax.experimental import pallas as pl
from jax.experimental.pallas import tpu as pltpu
from jax.experimental.pallas import tpu_sc as plsc

NU = 25000
NV = 25000
DIN = 128
H0 = 128
HALF = 64
FH = 64
EMB = 64
NSUP = 5
EPC = 80000
NPAIR = 100000
RB = 1000

ECH = 128
NCHUNK = NSUP * EPC // ECH
CPC = EPC // ECH
GSZ = 13
NGRP = 15
NLEFT = NCHUNK - 16 * NGRP * GSZ
CLR = 1000


def _proj_body(u_ref, v_ref, wg_ref, tu_ref, tv_ref):
    wacc = jnp.zeros((DIN, H0), jnp.float32)
    for r in range(NSUP):
        wacc = wacc + wg_ref[r]
        tu_ref[r] = jnp.dot(u_ref[...], wacc, preferred_element_type=jnp.float32)
        tv_ref[r] = jnp.dot(v_ref[...], wacc, preferred_element_type=jnp.float32)


def _project(u_features, v_features, W_gcn):
    nblk = NU // RB
    return pl.pallas_call(
        _proj_body,
        grid=(nblk,),
        in_specs=[
            pl.BlockSpec((RB, DIN), lambda j: (j, 0)),
            pl.BlockSpec((RB, DIN), lambda j: (j, 0)),
            pl.BlockSpec((NSUP, DIN, H0), lambda j: (0, 0, 0)),
        ],
        out_specs=[
            pl.BlockSpec((NSUP, RB, H0), lambda j: (0, j, 0)),
            pl.BlockSpec((NSUP, RB, H0), lambda j: (0, j, 0)),
        ],
        out_shape=[
            jax.ShapeDtypeStruct((NSUP, NU, H0), jnp.float32),
            jax.ShapeDtypeStruct((NSUP, NV, H0), jnp.float32),
        ],
    )(u_features, v_features, W_gcn)


def _mp_body(eu2_hbm, ev2_hbm, evals2_hbm, tmpu_hbm, tmpv_hbm, zeros_hbm,
             zu_out, zv_out,
             acc, rows_a, rows_b, idx_g2, idx_d2, evals2,
             semL, semG, semS):
    core = lax.axis_index("c")
    tid = lax.axis_index("s")

    def clear_acc():
        for k in range(2):
            c = tid + 16 * k

            @pl.when(c < NU // CLR)
            def _():
                pltpu.sync_copy(zeros_hbm.at[pl.ds(c * CLR, CLR)],
                                acc.at[pl.ds(c * CLR, CLR)])

    def dump(out_z):
        for k in range(2):
            c = tid + 16 * k

            @pl.when(c < NU // CLR)
            def _():
                @pl.when(core == 0)
                def _():
                    pltpu.sync_copy(acc.at[pl.ds(c * CLR, CLR)],
                                    out_z.at[pl.ds(c * CLR, CLR),
                                             pl.ds(0, HALF)])

                @pl.when(core == 1)
                def _():
                    pltpu.sync_copy(acc.at[pl.ds(c * CLR, CLR)],
                                    out_z.at[pl.ds(c * CLR, CLR),
                                             pl.ds(HALF, HALF)])

    def scale(cur, j):
        @plsc.parallel_loop(0, ECH // 16, unroll=2)
        def _(g):
            e16 = evals2[j, pl.ds(g * 16, 16)]
            for jj in range(16):
                s = e16.at[jnp.full((16,), jj, jnp.int32)].get(
                    mode="promise_in_bounds")
                e = g * 16 + jj
                for q in range(HALF // 16):
                    cur[e, pl.ds(q * 16, 16)] = cur[e, pl.ds(q * 16, 16)] * s

    def off_add(j, cglob):
        off = 2 * (cglob // CPC) * NU + core
        for g in range(ECH // 16):
            idx_g2[j, pl.ds(g * 16, 16)] = (
                idx_g2[j, pl.ds(g * 16, 16)] * 2 + off)

    def do_phase(tbl, gidx2, sidx2):
        def group(gq, cg):
            c0 = (tid * NGRP + gq) * GSZ
            h1 = pltpu.async_copy(gidx2.at[pl.ds(c0, GSZ)], idx_g2, semL)
            h2 = pltpu.async_copy(sidx2.at[pl.ds(c0, GSZ)], idx_d2, semL)
            h3 = pltpu.async_copy(evals2_hbm.at[pl.ds(c0, GSZ)], evals2, semL)
            h1.wait()
            h2.wait()
            h3.wait()

            def oadd(j, c):
                off_add(j, c0 + j)
                return c
            lax.fori_loop(0, GSZ, oadd, 0)

            pltpu.async_copy(tbl.at[idx_g2.at[0]], rows_a, semG)

            def run(j, cur, oth):
                pltpu.make_async_copy(tbl.at[idx_g2.at[j]], cur, semG).wait()

                @pl.when(j > 0)
                def _():
                    pltpu.make_async_copy(
                        oth, acc.at[idx_d2.at[j - 1]], semS).wait()

                @pl.when(j < GSZ - 1)
                def _():
                    pltpu.async_copy(tbl.at[idx_g2.at[j + 1]], oth, semG)

                scale(cur, j)
                pltpu.async_copy(cur, acc.at[idx_d2.at[j]], semS, add=True)

            def chunk_j(j, c):
                @pl.when(j % 2 == 0)
                def _():
                    run(j, rows_a, rows_b)

                @pl.when(j % 2 == 1)
                def _():
                    run(j, rows_b, rows_a)
                return c
            lax.fori_loop(0, GSZ, chunk_j, 0)
            pltpu.make_async_copy(
                rows_a, acc.at[idx_d2.at[GSZ - 1]], semS).wait()
            return cg
        lax.fori_loop(0, NGRP, group, 0)

        @pl.when(tid < NLEFT)
        def _():
            row = 16 * NGRP * GSZ + tid
            pltpu.sync_copy(gidx2.at[pl.ds(row, 1)], idx_g2.at[pl.ds(0, 1)])
            pltpu.sync_copy(sidx2.at[pl.ds(row, 1)], idx_d2.at[pl.ds(0, 1)])
            pltpu.sync_copy(evals2_hbm.at[pl.ds(row, 1)],
                            evals2.at[pl.ds(0, 1)])
            off_add(0, row)
            pltpu.async_copy(tbl.at[idx_g2.at[0]], rows_a, semG).wait()
            scale(rows_a, 0)
            pltpu.sync_copy(rows_a, acc.at[idx_d2.at[0]], add=True)

    clear_acc()
    plsc.subcore_barrier()
    do_phase(tmpv_hbm, ev2_hbm, eu2_hbm)
    plsc.subcore_barrier()
    dump(zu_out)
    plsc.subcore_barrier()
    clear_acc()
    plsc.subcore_barrier()
    do_phase(tmpu_hbm, eu2_hbm, ev2_hbm)
    plsc.subcore_barrier()
    dump(zv_out)


def _message_passing(eu, ev, evals, tmp_u_flat, tmp_v_flat):
    mesh = plsc.VectorSubcoreMesh(core_axis_name="c", subcore_axis_name="s")
    zshape = jax.ShapeDtypeStruct((NU, H0), jnp.float32)
    eu2 = eu.reshape(NCHUNK, ECH)
    ev2 = ev.reshape(NCHUNK, ECH)
    evals2d = evals.reshape(NCHUNK, ECH)
    zeros = jnp.zeros((NU, HALF), jnp.float32)
    return pl.kernel(
        _mp_body,
        out_type=[zshape, zshape],
        mesh=mesh,
        scratch_types=[
            pltpu.VMEM_SHARED((NU, HALF), jnp.float32),
            pltpu.VMEM((ECH, HALF), jnp.float32),
            pltpu.VMEM((ECH, HALF), jnp.float32),
            pltpu.VMEM((GSZ, ECH), jnp.int32),
            pltpu.VMEM((GSZ, ECH), jnp.int32),
            pltpu.VMEM((GSZ, ECH), jnp.float32),
            pltpu.SemaphoreType.DMA,
            pltpu.SemaphoreType.DMA,
            pltpu.SemaphoreType.DMA,
        ],
        compiler_params=pltpu.CompilerParams(use_tc_tiling_on_sc=False),
    )(eu2, ev2, evals2d, tmp_u_flat, tmp_v_flat, zeros)


def _comb_body(zu, zv, su, sv, wfu, bfu, wfv, bfv, wdu, wdv, out_ref):
    def side(s_ref, wf, bf):
        t = jnp.dot(s_ref[...], wf[...], preferred_element_type=jnp.float32)
        return jnp.maximum(t + bf[...], 0.0)

    def emb(z, f, wd):
        w = wd[...]
        return (jnp.dot(jnp.maximum(z[...], 0.0), w[:H0],
                        preferred_element_type=jnp.float32)
                + jnp.dot(f, w[H0:], preferred_element_type=jnp.float32))

    out_ref[:, :EMB] = emb(zu, side(su, wfu, bfu), wdu)
    out_ref[:, EMB:] = emb(zv, side(sv, wfv, bfv), wdv)


def _combine(zu, zv, su, sv, wfu, bfu, wfv, bfv, wdu, wdv):
    nblk = NU // RB
    zspec = pl.BlockSpec((RB, H0), lambda j: (j, 0))
    sspec = pl.BlockSpec((RB, 32), lambda j: (j, 0))
    wfspec = pl.BlockSpec((32, FH), lambda j: (0, 0))
    bspec = pl.BlockSpec((1, FH), lambda j: (0, 0))
    wdspec = pl.BlockSpec((H0 + FH, EMB), lambda j: (0, 0))
    return pl.pallas_call(
        _comb_body,
        grid=(nblk,),
        in_specs=[zspec, zspec, sspec, sspec,
                  wfspec, bspec, wfspec, bspec, wdspec, wdspec],
        out_specs=pl.BlockSpec((RB, 2 * EMB), lambda j: (j, 0)),
        out_shape=jax.ShapeDtypeStruct((NU, 2 * EMB), jnp.float32),
    )(zu, zv, su, sv, wfu, bfu.reshape(1, FH), wfv, bfv.reshape(1, FH),
      wdu, wdv)


PCH = 80
NPCH = NPAIR // PCH


def _pairs_body(uidx_hbm, vidx_hbm, emb2_hbm, out_hbm,
                idxu, idxv, buf_u, buf_v, semL, semG):
    core = lax.axis_index("c")
    tid = lax.axis_index("s")
    wid = tid * 2 + core
    nk = 39 + jnp.where(wid < NPCH - 39 * 32, 1, 0)

    def chunk(kk, c):
        cidx = wid + 32 * kk
        base = cidx * PCH
        h1 = pltpu.async_copy(uidx_hbm.at[pl.ds(base, PCH)], idxu, semL)
        h2 = pltpu.async_copy(vidx_hbm.at[pl.ds(base, PCH)], idxv, semL)
        h1.wait()
        h2.wait()
        for g in range(PCH // 16):
            idxu[pl.ds(g * 16, 16)] = idxu[pl.ds(g * 16, 16)] * 2
            idxv[pl.ds(g * 16, 16)] = idxv[pl.ds(g * 16, 16)] * 2 + 1
        hu = pltpu.async_copy(emb2_hbm.at[idxu], buf_u, semG)
        hv = pltpu.async_copy(emb2_hbm.at[idxv], buf_v, semG)
        hu.wait()
        pltpu.sync_copy(buf_u, out_hbm.at[pl.ds(base, PCH), pl.ds(0, EMB)])
        hv.wait()
        pltpu.sync_copy(buf_v, out_hbm.at[pl.ds(base, PCH), pl.ds(EMB, EMB)])
        return c
    lax.fori_loop(0, nk, chunk, 0)


def _gather_pairs(ui, vi, emb_cat):
    mesh = plsc.VectorSubcoreMesh(core_axis_name="c", subcore_axis_name="s")
    emb2 = emb_cat.reshape(2 * NU, EMB)
    return pl.kernel(
        _pairs_body,
        out_type=jax.ShapeDtypeStruct((NPAIR, 2 * EMB), jnp.float32),
        mesh=mesh,
        scratch_types=[
            pltpu.VMEM((PCH,), jnp.int32),
            pltpu.VMEM((PCH,), jnp.int32),
            pltpu.VMEM((PCH, EMB), jnp.float32),
            pltpu.VMEM((PCH, EMB), jnp.float32),
            pltpu.SemaphoreType.DMA,
            pltpu.SemaphoreType.DMA,
        ],
        compiler_params=pltpu.CompilerParams(use_tc_tiling_on_sc=False),
    )(ui, vi, emb2)


PB = 2000


def _dec_body(g_ref, p_ref, wc_ref, out_ref):
    gu = g_ref[:, :EMB]
    gv = g_ref[:, EMB:]
    b0 = jnp.sum(jnp.dot(gu, p_ref[0], preferred_element_type=jnp.float32)
                 * gv, axis=1)
    b1 = jnp.sum(jnp.dot(gu, p_ref[1], preferred_element_type=jnp.float32)
                 * gv, axis=1)
    out_ref[...] = (b0[:, None] * wc_ref[0][None, :]
                    + b1[:, None] * wc_ref[1][None, :])


def _decode(gugv, P_basis, W_comb):
    nblk = NPAIR // PB
    ncls = W_comb.shape[1]
    return pl.pallas_call(
        _dec_body,
        grid=(nblk,),
        in_specs=[
            pl.BlockSpec((PB, 2 * EMB), lambda j: (j, 0)),
            pl.BlockSpec((2, EMB, EMB), lambda j: (0, 0, 0)),
            pl.BlockSpec((2, ncls), lambda j: (0, 0)),
        ],
        out_specs=pl.BlockSpec((PB, ncls), lambda j: (j, 0)),
        out_shape=jax.ShapeDtypeStruct((NPAIR, ncls), jnp.float32),
    )(gugv, P_basis, W_comb)


def kernel(u_features, v_features, u_features_side, v_features_side,
           edge_u, edge_v, edge_vals, u_indices, v_indices,
           W_gcn, W_feat_u, b_feat_u, W_feat_v, b_feat_v,
           W_dense_u, W_dense_v, P_basis, W_comb):
    eu = edge_u.astype(jnp.int32)
    ev = edge_v.astype(jnp.int32)
    ui = u_indices.astype(jnp.int32)
    vi = v_indices.astype(jnp.int32)

    tmp_u, tmp_v = _project(u_features, v_features, W_gcn)
    tmp_u_flat = tmp_u.reshape(2 * NSUP * NU, HALF)
    tmp_v_flat = tmp_v.reshape(2 * NSUP * NV, HALF)

    zu, zv = _message_passing(eu, ev, edge_vals, tmp_u_flat, tmp_v_flat)

    emb_cat = _combine(zu, zv, u_features_side, v_features_side,
                       W_feat_u, b_feat_u, W_feat_v, b_feat_v,
                       W_dense_u, W_dense_v)

    gugv = _gather_pairs(ui, vi, emb_cat)
    return _decode(gugv, P_basis, W_comb)


# speedup vs baseline: 2.0364x; 1.1457x over previous
import jax
import jax.numpy as jnp
from jax import lax
from jax.experimental import pallas as pl
from jax.experimental.pallas import tpu as pltpu
from jax.experimental.pallas import tpu_sc as plsc

NU = 25000
NV = 25000
DIN = 128
H0 = 128
HALF = 64
FH = 64
EMB = 64
NSUP = 5
EPC = 80000
NPAIR = 100000
RB = 1000

ECH = 128
NCHUNK = NSUP * EPC // ECH
CPC = EPC // ECH
GSZ = 13
NGRP = 15
NLEFT = NCHUNK - 16 * NGRP * GSZ
CLR = 1000


def _proj_body(u_ref, v_ref, wg_ref, tu_ref, tv_ref):
    wacc = jnp.zeros((DIN, H0), jnp.float32)
    for r in range(NSUP):
        wacc = wacc + wg_ref[r]
        tu_ref[r] = jnp.dot(u_ref[...], wacc, preferred_element_type=jnp.float32)
        tv_ref[r] = jnp.dot(v_ref[...], wacc, preferred_element_type=jnp.float32)


def _project(u_features, v_features, W_gcn):
    nblk = NU // RB
    return pl.pallas_call(
        _proj_body,
        grid=(nblk,),
        in_specs=[
            pl.BlockSpec((RB, DIN), lambda j: (j, 0)),
            pl.BlockSpec((RB, DIN), lambda j: (j, 0)),
            pl.BlockSpec((NSUP, DIN, H0), lambda j: (0, 0, 0)),
        ],
        out_specs=[
            pl.BlockSpec((NSUP, RB, H0), lambda j: (0, j, 0)),
            pl.BlockSpec((NSUP, RB, H0), lambda j: (0, j, 0)),
        ],
        out_shape=[
            jax.ShapeDtypeStruct((NSUP, NU, H0), jnp.float32),
            jax.ShapeDtypeStruct((NSUP, NV, H0), jnp.float32),
        ],
    )(u_features, v_features, W_gcn)


def _mp_body(eu2_hbm, ev2_hbm, evals2_hbm, tmpu_hbm, tmpv_hbm, zeros_hbm,
             zu_out, zv_out,
             acc, rows_a, rows_b, rows_c, idx_g2, idx_d2, evals2,
             semL, semG, semS):
    core = lax.axis_index("c")
    tid = lax.axis_index("s")

    def clear_acc():
        for k in range(2):
            c = tid + 16 * k

            @pl.when(c < NU // CLR)
            def _():
                pltpu.sync_copy(zeros_hbm.at[pl.ds(c * CLR, CLR)],
                                acc.at[pl.ds(c * CLR, CLR)])

    def dump(out_z):
        for k in range(2):
            c = tid + 16 * k

            @pl.when(c < NU // CLR)
            def _():
                @pl.when(core == 0)
                def _():
                    pltpu.sync_copy(acc.at[pl.ds(c * CLR, CLR)],
                                    out_z.at[pl.ds(c * CLR, CLR),
                                             pl.ds(0, HALF)])

                @pl.when(core == 1)
                def _():
                    pltpu.sync_copy(acc.at[pl.ds(c * CLR, CLR)],
                                    out_z.at[pl.ds(c * CLR, CLR),
                                             pl.ds(HALF, HALF)])

    def scale(cur, j):
        @plsc.parallel_loop(0, ECH // 16, unroll=2)
        def _(g):
            e16 = evals2[j, pl.ds(g * 16, 16)]
            for jj in range(16):
                s = e16.at[jnp.full((16,), jj, jnp.int32)].get(
                    mode="promise_in_bounds")
                e = g * 16 + jj
                for q in range(HALF // 16):
                    cur[e, pl.ds(q * 16, 16)] = cur[e, pl.ds(q * 16, 16)] * s

    def off_add(j, cglob):
        off = 2 * (cglob // CPC) * NU + core
        for g in range(ECH // 16):
            idx_g2[j, pl.ds(g * 16, 16)] = (
                idx_g2[j, pl.ds(g * 16, 16)] * 2 + off)

    def do_phase(tbl, gidx2, sidx2):
        def group(gq, cg):
            c0 = (tid * NGRP + gq) * GSZ
            h1 = pltpu.async_copy(gidx2.at[pl.ds(c0, GSZ)], idx_g2, semL)
            h2 = pltpu.async_copy(sidx2.at[pl.ds(c0, GSZ)], idx_d2, semL)
            h3 = pltpu.async_copy(evals2_hbm.at[pl.ds(c0, GSZ)], evals2, semL)
            h1.wait()
            h2.wait()
            h3.wait()

            def oadd(j, c):
                off_add(j, c0 + j)
                return c
            lax.fori_loop(0, GSZ, oadd, 0)

            pltpu.async_copy(tbl.at[idx_g2.at[0]], rows_a, semG)
            pltpu.async_copy(tbl.at[idx_g2.at[1]], rows_b, semG)

            def run(j, cur, nxt2):
                pltpu.make_async_copy(tbl.at[idx_g2.at[j]], cur, semG).wait()

                @pl.when(j > 0)
                def _():
                    pltpu.make_async_copy(
                        nxt2, acc.at[idx_d2.at[j - 1]], semS).wait()

                @pl.when(j < GSZ - 2)
                def _():
                    pltpu.async_copy(tbl.at[idx_g2.at[j + 2]], nxt2, semG)

                scale(cur, j)
                pltpu.async_copy(cur, acc.at[idx_d2.at[j]], semS, add=True)

            def chunk_j(j, c):
                @pl.when(j % 3 == 0)
                def _():
                    run(j, rows_a, rows_c)

                @pl.when(j % 3 == 1)
                def _():
                    run(j, rows_b, rows_a)

                @pl.when(j % 3 == 2)
                def _():
                    run(j, rows_c, rows_b)
                return c
            lax.fori_loop(0, GSZ, chunk_j, 0)
            pltpu.make_async_copy(
                rows_a, acc.at[idx_d2.at[GSZ - 1]], semS).wait()
            return cg
        lax.fori_loop(0, NGRP, group, 0)

        @pl.when(tid < NLEFT)
        def _():
            row = 16 * NGRP * GSZ + tid
            pltpu.sync_copy(gidx2.at[pl.ds(row, 1)], idx_g2.at[pl.ds(0, 1)])
            pltpu.sync_copy(sidx2.at[pl.ds(row, 1)], idx_d2.at[pl.ds(0, 1)])
            pltpu.sync_copy(evals2_hbm.at[pl.ds(row, 1)],
                            evals2.at[pl.ds(0, 1)])
            off_add(0, row)
            pltpu.async_copy(tbl.at[idx_g2.at[0]], rows_a, semG).wait()
            scale(rows_a, 0)
            pltpu.sync_copy(rows_a, acc.at[idx_d2.at[0]], add=True)

    clear_acc()
    plsc.subcore_barrier()
    do_phase(tmpv_hbm, ev2_hbm, eu2_hbm)
    plsc.subcore_barrier()
    dump(zu_out)
    plsc.subcore_barrier()
    clear_acc()
    plsc.subcore_barrier()
    do_phase(tmpu_hbm, eu2_hbm, ev2_hbm)
    plsc.subcore_barrier()
    dump(zv_out)


def _message_passing(eu, ev, evals, tmp_u_flat, tmp_v_flat):
    mesh = plsc.VectorSubcoreMesh(core_axis_name="c", subcore_axis_name="s")
    zshape = jax.ShapeDtypeStruct((NU, H0), jnp.float32)
    eu2 = eu.reshape(NCHUNK, ECH)
    ev2 = ev.reshape(NCHUNK, ECH)
    evals2d = evals.reshape(NCHUNK, ECH)
    zeros = jnp.zeros((NU, HALF), jnp.float32)
    return pl.kernel(
        _mp_body,
        out_type=[zshape, zshape],
        mesh=mesh,
        scratch_types=[
            pltpu.VMEM_SHARED((NU, HALF), jnp.float32),
            pltpu.VMEM((ECH, HALF), jnp.float32),
            pltpu.VMEM((ECH, HALF), jnp.float32),
            pltpu.VMEM((ECH, HALF), jnp.float32),
            pltpu.VMEM((GSZ, ECH), jnp.int32),
            pltpu.VMEM((GSZ, ECH), jnp.int32),
            pltpu.VMEM((GSZ, ECH), jnp.float32),
            pltpu.SemaphoreType.DMA,
            pltpu.SemaphoreType.DMA,
            pltpu.SemaphoreType.DMA,
        ],
        compiler_params=pltpu.CompilerParams(use_tc_tiling_on_sc=False),
    )(eu2, ev2, evals2d, tmp_u_flat, tmp_v_flat, zeros)


def _comb_body(zu, zv, su, sv, wfu, bfu, wfv, bfv, wdu, wdv, out_ref):
    def side(s_ref, wf, bf):
        t = jnp.dot(s_ref[...], wf[...], preferred_element_type=jnp.float32)
        return jnp.maximum(t + bf[...], 0.0)

    def emb(z, f, wd):
        w = wd[...]
        return (jnp.dot(jnp.maximum(z[...], 0.0), w[:H0],
                        preferred_element_type=jnp.float32)
                + jnp.dot(f, w[H0:], preferred_element_type=jnp.float32))

    out_ref[:, :EMB] = emb(zu, side(su, wfu, bfu), wdu)
    out_ref[:, EMB:] = emb(zv, side(sv, wfv, bfv), wdv)


def _combine(zu, zv, su, sv, wfu, bfu, wfv, bfv, wdu, wdv):
    nblk = NU // RB
    zspec = pl.BlockSpec((RB, H0), lambda j: (j, 0))
    sspec = pl.BlockSpec((RB, 32), lambda j: (j, 0))
    wfspec = pl.BlockSpec((32, FH), lambda j: (0, 0))
    bspec = pl.BlockSpec((1, FH), lambda j: (0, 0))
    wdspec = pl.BlockSpec((H0 + FH, EMB), lambda j: (0, 0))
    return pl.pallas_call(
        _comb_body,
        grid=(nblk,),
        in_specs=[zspec, zspec, sspec, sspec,
                  wfspec, bspec, wfspec, bspec, wdspec, wdspec],
        out_specs=pl.BlockSpec((RB, 2 * EMB), lambda j: (j, 0)),
        out_shape=jax.ShapeDtypeStruct((NU, 2 * EMB), jnp.float32),
    )(zu, zv, su, sv, wfu, bfu.reshape(1, FH), wfv, bfv.reshape(1, FH),
      wdu, wdv)


PCH = 80
NPCH = NPAIR // PCH


def _pairs_body(uidx_hbm, vidx_hbm, emb2_hbm, out_hbm,
                idxu, idxv, buf_u, buf_v, semL, semG):
    core = lax.axis_index("c")
    tid = lax.axis_index("s")
    wid = tid * 2 + core
    nk = 39 + jnp.where(wid < NPCH - 39 * 32, 1, 0)

    def chunk(kk, c):
        cidx = wid + 32 * kk
        base = cidx * PCH
        h1 = pltpu.async_copy(uidx_hbm.at[pl.ds(base, PCH)], idxu, semL)
        h2 = pltpu.async_copy(vidx_hbm.at[pl.ds(base, PCH)], idxv, semL)
        h1.wait()
        h2.wait()
        for g in range(PCH // 16):
            idxu[pl.ds(g * 16, 16)] = idxu[pl.ds(g * 16, 16)] * 2
            idxv[pl.ds(g * 16, 16)] = idxv[pl.ds(g * 16, 16)] * 2 + 1
        hu = pltpu.async_copy(emb2_hbm.at[idxu], buf_u, semG)
        hv = pltpu.async_copy(emb2_hbm.at[idxv], buf_v, semG)
        hu.wait()
        pltpu.sync_copy(buf_u, out_hbm.at[pl.ds(base, PCH), pl.ds(0, EMB)])
        hv.wait()
        pltpu.sync_copy(buf_v, out_hbm.at[pl.ds(base, PCH), pl.ds(EMB, EMB)])
        return c
    lax.fori_loop(0, nk, chunk, 0)


def _gather_pairs(ui, vi, emb_cat):
    mesh = plsc.VectorSubcoreMesh(core_axis_name="c", subcore_axis_name="s")
    emb2 = emb_cat.reshape(2 * NU, EMB)
    return pl.kernel(
        _pairs_body,
        out_type=jax.ShapeDtypeStruct((NPAIR, 2 * EMB), jnp.float32),
        mesh=mesh,
        scratch_types=[
            pltpu.VMEM((PCH,), jnp.int32),
            pltpu.VMEM((PCH,), jnp.int32),
            pltpu.VMEM((PCH, EMB), jnp.float32),
            pltpu.VMEM((PCH, EMB), jnp.float32),
            pltpu.SemaphoreType.DMA,
            pltpu.SemaphoreType.DMA,
        ],
        compiler_params=pltpu.CompilerParams(use_tc_tiling_on_sc=False),
    )(ui, vi, emb2)


PB = 2000


def _dec_body(g_ref, p_ref, wc_ref, out_ref):
    gu = g_ref[:, :EMB]
    gv = g_ref[:, EMB:]
    b0 = jnp.sum(jnp.dot(gu, p_ref[0], preferred_element_type=jnp.float32)
                 * gv, axis=1)
    b1 = jnp.sum(jnp.dot(gu, p_ref[1], preferred_element_type=jnp.float32)
                 * gv, axis=1)
    out_ref[...] = (b0[:, None] * wc_ref[0][None, :]
                    + b1[:, None] * wc_ref[1][None, :])


def _decode(gugv, P_basis, W_comb):
    nblk = NPAIR // PB
    ncls = W_comb.shape[1]
    return pl.pallas_call(
        _dec_body,
        grid=(nblk,),
        in_specs=[
            pl.BlockSpec((PB, 2 * EMB), lambda j: (j, 0)),
            pl.BlockSpec((2, EMB, EMB), lambda j: (0, 0, 0)),
            pl.BlockSpec((2, ncls), lambda j: (0, 0)),
        ],
        out_specs=pl.BlockSpec((PB, ncls), lambda j: (j, 0)),
        out_shape=jax.ShapeDtypeStruct((NPAIR, ncls), jnp.float32),
    )(gugv, P_basis, W_comb)


def kernel(u_features, v_features, u_features_side, v_features_side,
           edge_u, edge_v, edge_vals, u_indices, v_indices,
           W_gcn, W_feat_u, b_feat_u, W_feat_v, b_feat_v,
           W_dense_u, W_dense_v, P_basis, W_comb):
    eu = edge_u.astype(jnp.int32)
    ev = edge_v.astype(jnp.int32)
    ui = u_indices.astype(jnp.int32)
    vi = v_indices.astype(jnp.int32)

    tmp_u, tmp_v = _project(u_features, v_features, W_gcn)
    tmp_u_flat = tmp_u.reshape(2 * NSUP * NU, HALF)
    tmp_v_flat = tmp_v.reshape(2 * NSUP * NV, HALF)

    zu, zv = _message_passing(eu, ev, edge_vals, tmp_u_flat, tmp_v_flat)

    emb_cat = _combine(zu, zv, u_features_side, v_features_side,
                       W_feat_u, b_feat_u, W_feat_v, b_feat_v,
                       W_dense_u, W_dense_v)

    gugv = _gather_pairs(ui, vi, emb_cat)
    return _decode(gugv, P_basis, W_comb)
